# Initial kernel scaffold; baseline (speedup 1.0000x reference)
import jax
import jax.numpy as jnp
from jax.experimental import pallas as pl


def kernel(points, features, frames, params):
    raise NotImplementedError("write your pallas kernel here")


# speedup vs baseline: 338.4410x; 338.4410x over previous
import functools

import jax
import jax.numpy as jnp
from jax import lax
from jax.experimental import pallas as pl
from jax.experimental.pallas import tpu as pltpu
from jax.experimental.pallas import tpu_sc as plsc

_EPS = 1e-5
_K = 7
_F32 = jnp.float32


def _dot(a, b):
    return lax.dot_general(a, b, (((1,), (0,)), ((), ())),
                           preferred_element_type=_F32)


def _dot_t(a, b):
    return lax.dot_general(a, b, (((1,), (1,)), ((), ())),
                           preferred_element_type=_F32)


def _prep_body(P, p_ref, f_ref, fr_ref, pts_o, fm_o, cs_o, f16_o,
               s_o, q_o, cnt_o):
    b = pl.program_id(0)
    f = f_ref[0]
    absum = jnp.sum(jnp.abs(f), axis=0, keepdims=True)
    mask = (absum != 0.0).astype(_F32)
    fm = f * mask
    fm_o[0] = fm
    pts_o[0] = p_ref[0] * mask
    cs_o[0] = (1.0 - mask) * 1e9
    f16_o[0] = jnp.transpose(fr_ref[0], (1, 0))
    cnt_o[0] = jnp.maximum(jnp.sum(mask, axis=1, keepdims=True), 1.0)
    ps = jnp.sum(fm, axis=1, keepdims=True)
    pq = jnp.sum(fm * fm, axis=1, keepdims=True)

    @pl.when(b == 0)
    def _():
        s_o[...] = ps
        q_o[...] = pq

    @pl.when(b != 0)
    def _():
        s_o[...] += ps
        q_o[...] += pq


def _prep(points, features, frames_r):
    B, _, P = points.shape
    return pl.pallas_call(
        functools.partial(_prep_body, P),
        grid=(B,),
        in_specs=[
            pl.BlockSpec((1, 3, P), lambda b: (b, 0, 0)),
            pl.BlockSpec((1, 8, P), lambda b: (b, 0, 0)),
            pl.BlockSpec((1, P, 16), lambda b: (b, 0, 0)),
        ],
        out_specs=[
            pl.BlockSpec((1, 3, P), lambda b: (b, 0, 0)),
            pl.BlockSpec((1, 8, P), lambda b: (b, 0, 0)),
            pl.BlockSpec((1, 1, P), lambda b: (b, 0, 0)),
            pl.BlockSpec((1, 16, P), lambda b: (b, 0, 0)),
            pl.BlockSpec((8, 1), lambda b: (0, 0)),
            pl.BlockSpec((8, 1), lambda b: (0, 0)),
            pl.BlockSpec((1, 1, 1), lambda b: (b, 0, 0)),
        ],
        out_shape=[
            jax.ShapeDtypeStruct((B, 3, P), _F32),
            jax.ShapeDtypeStruct((B, 8, P), _F32),
            jax.ShapeDtypeStruct((B, 1, P), _F32),
            jax.ShapeDtypeStruct((B, 16, P), _F32),
            jax.ShapeDtypeStruct((8, 1), _F32),
            jax.ShapeDtypeStruct((8, 1), _F32),
            jax.ShapeDtypeStruct((B, 1, 1), _F32),
        ],
    )(points, features, frames_r)


def _knn_body(P, Dp, Df, nv, Dy, apply_bn, n_bn, *refs):
    if apply_bn:
        (pts_ref, cs_ref, f_ref, f16_ref, s_ref, q_ref, g_ref, b_ref,
         idx_o, y_o, fts_o) = refs
    else:
        pts_ref, cs_ref, f_ref, f16_ref, idx_o, y_o = refs
    b = pl.program_id(0)
    x = pts_ref[0] + cs_ref[0]
    if apply_bn:
        m = s_ref[...] / n_bn
        var = q_ref[...] / n_bn - m * m
        mask = (cs_ref[0] == 0.0).astype(_F32)
        fts = (g_ref[...] * (f_ref[0] - m) / jnp.sqrt(var + _EPS)
               + b_ref[...]) * mask
        fts_o[0] = fts
    else:
        fts = f_ref[0]

    inner = -2.0 * lax.dot_general(x, x, (((0,), (0,)), ((), ())),
                                   preferred_element_type=_F32)
    xx = jnp.sum(x * x, axis=0, keepdims=True)
    xxc = jnp.transpose(xx, (1, 0))
    pd = -xxc - inner - xx

    riota = lax.broadcasted_iota(jnp.int32, (P, P), 0)
    vals = pd
    rows = []
    for t in range(_K + 1):
        mx = jnp.max(vals, axis=0, keepdims=True)
        cand = jnp.where(vals == mx, riota, P)
        am = jnp.min(cand, axis=0, keepdims=True)
        if t > 0:
            rows.append(am)
        if t < _K:
            vals = jnp.where(riota == am, -jnp.inf, vals)
    idx_o[0] = jnp.concatenate(rows, axis=0) + b * P

    urows = []
    for c in range(4):
        for v in range(nv):
            u = f16_ref[0, c:c + 1, :] * fts[4 * v:4 * v + 1, :]
            for a in range(1, 4):
                u += (f16_ref[0, a * 4 + c:a * 4 + c + 1, :]
                      * fts[4 * v + a:4 * v + a + 1, :])
            urows.append(u)
    npad = Dy - 4 * nv
    if npad:
        urows.append(jnp.zeros((npad, P), _F32))
    y_o[0] = jnp.transpose(jnp.concatenate(urows, axis=0), (1, 0))


def _knn(pts, cshift, feats, f16, nv, Dy, bn=None):
    B, Dp, P = pts.shape
    Df = feats.shape[1]
    apply_bn = bn is not None
    ins = [pts, cshift, feats, f16]
    in_specs = [
        pl.BlockSpec((1, Dp, P), lambda b: (b, 0, 0)),
        pl.BlockSpec((1, 1, P), lambda b: (b, 0, 0)),
        pl.BlockSpec((1, Df, P), lambda b: (b, 0, 0)),
        pl.BlockSpec((1, 16, P), lambda b: (b, 0, 0)),
    ]
    out_specs = [
        pl.BlockSpec((1, _K, P), lambda b: (b, 0, 0)),
        pl.BlockSpec((1, P, Dy), lambda b: (b, 0, 0)),
    ]
    out_shape = [
        jax.ShapeDtypeStruct((B, _K, P), jnp.int32),
        jax.ShapeDtypeStruct((B, P, Dy), _F32),
    ]
    n_bn = 0.0
    if apply_bn:
        s, q, g, bb, n_bn = bn
        ins += [s, q, g, bb]
        in_specs += [pl.BlockSpec((Df, 1), lambda b: (0, 0))] * 4
        out_specs.append(pl.BlockSpec((1, Df, P), lambda b: (b, 0, 0)))
        out_shape.append(jax.ShapeDtypeStruct((B, Df, P), _F32))
    return pl.pallas_call(
        functools.partial(_knn_body, P, Dp, Df, nv, Dy, apply_bn, n_bn),
        grid=(B,), in_specs=in_specs, out_specs=out_specs,
        out_shape=out_shape,
    )(*ins)


def _sc_gather(table, idx_flat, D):
    E = idx_flat.shape[0]
    NW = 32
    per_w = E // NW
    chunk = 3584 if D <= 16 else 1792
    nch = per_w // chunk
    mesh = plsc.VectorSubcoreMesh(core_axis_name="c", subcore_axis_name="s")

    @functools.partial(
        pl.kernel, mesh=mesh,
        out_type=jax.ShapeDtypeStruct((E, D), _F32),
        compiler_params=pltpu.CompilerParams(use_tc_tiling_on_sc=False),
        scratch_types=[
            pltpu.VMEM((chunk,), jnp.int32),
            pltpu.VMEM((chunk, D), _F32),
            pltpu.SemaphoreType.DMA,
        ],
    )
    def gk(table_h, idx_h, out_h, idx_v, rows_v, sem):
        wid = lax.axis_index("s") * 2 + lax.axis_index("c")
        base = wid * per_w
        for ci in range(nch):
            off = base + ci * chunk
            pltpu.sync_copy(idx_h.at[pl.ds(off, chunk)], idx_v)
            pltpu.async_copy(table_h.at[idx_v], rows_v, sem).wait()
            pltpu.sync_copy(rows_v, out_h.at[pl.ds(off, chunk)])

    return gk(table, idx_flat)


def _conv1_body(P, nv, Dy, O, f_ref, f16_ref, yg_ref, A_ref, Bz_ref, scW_ref,
                y_o, sc_o, s_o, q_o, ss_o, sq_o):
    b = pl.program_id(0)
    x = f_ref[0]
    ygT = jnp.transpose(yg_ref[0], (1, 0))
    base = _dot(A_ref[...], x)
    sc = _dot(scW_ref[...], x)
    sc_o[0] = sc
    s_acc = jnp.zeros((O, 1), _F32)
    q_acc = jnp.zeros((O, 1), _F32)
    for k in range(_K):
        ygk = ygT[:, k * P:(k + 1) * P]
        zr = []
        for a in range(4):
            za = f16_ref[0, a * 4:a * 4 + 1, :] * ygk[0:nv, :]
            for c in range(1, 4):
                za += (f16_ref[0, a * 4 + c:a * 4 + c + 1, :]
                       * ygk[c * nv:(c + 1) * nv, :])
            zr.append(za)
        z = jnp.concatenate(zr, axis=0)
        yk = base + _dot(Bz_ref[...], z)
        y_o[0, k] = yk
        s_acc += jnp.sum(yk, axis=1, keepdims=True)
        q_acc += jnp.sum(yk * yk, axis=1, keepdims=True)
    ssp = jnp.sum(sc, axis=1, keepdims=True)
    sqp = jnp.sum(sc * sc, axis=1, keepdims=True)

    @pl.when(b == 0)
    def _():
        s_o[...] = s_acc
        q_o[...] = q_acc
        ss_o[...] = ssp
        sq_o[...] = sqp

    @pl.when(b != 0)
    def _():
        s_o[...] += s_acc
        q_o[...] += q_acc
        ss_o[...] += ssp
        sq_o[...] += sqp


def _conv1(fts, f16, yg, A, Bz, scW, nv):
    B, Din, P = fts.shape
    Dy = yg.shape[-1]
    O = A.shape[0]
    return pl.pallas_call(
        functools.partial(_conv1_body, P, nv, Dy, O),
        grid=(B,),
        in_specs=[
            pl.BlockSpec((1, Din, P), lambda b: (b, 0, 0)),
            pl.BlockSpec((1, 16, P), lambda b: (b, 0, 0)),
            pl.BlockSpec((1, _K * P, Dy), lambda b: (b, 0, 0)),
            pl.BlockSpec((O, Din), lambda b: (0, 0)),
            pl.BlockSpec((O, Din), lambda b: (0, 0)),
            pl.BlockSpec((O, Din), lambda b: (0, 0)),
        ],
        out_specs=[
            pl.BlockSpec((1, _K, O, P), lambda b: (b, 0, 0, 0)),
            pl.BlockSpec((1, O, P), lambda b: (b, 0, 0)),
            pl.BlockSpec((O, 1), lambda b: (0, 0)),
            pl.BlockSpec((O, 1), lambda b: (0, 0)),
            pl.BlockSpec((O, 1), lambda b: (0, 0)),
            pl.BlockSpec((O, 1), lambda b: (0, 0)),
        ],
        out_shape=[
            jax.ShapeDtypeStruct((B, _K, O, P), _F32),
            jax.ShapeDtypeStruct((B, O, P), _F32),
            jax.ShapeDtypeStruct((O, 1), _F32),
            jax.ShapeDtypeStruct((O, 1), _F32),
            jax.ShapeDtypeStruct((O, 1), _F32),
            jax.ShapeDtypeStruct((O, 1), _F32),
        ],
    )(fts, f16, yg, A, Bz, scW)


def _convmid_body(P, O, O2, n, y_ref, s_ref, q_ref, g_ref, b_ref, W_ref,
                  y_o, s_o, q_o):
    b = pl.program_id(0)
    m = s_ref[...] / n
    var = q_ref[...] / n - m * m
    inv = 1.0 / jnp.sqrt(var + _EPS)
    s_acc = jnp.zeros((O2, 1), _F32)
    q_acc = jnp.zeros((O2, 1), _F32)
    for k in range(_K):
        yk = y_ref[0, k]
        xn = jnp.maximum(g_ref[...] * (yk - m) * inv + b_ref[...], 0.0)
        ok = _dot(W_ref[...], xn)
        y_o[0, k] = ok
        s_acc += jnp.sum(ok, axis=1, keepdims=True)
        q_acc += jnp.sum(ok * ok, axis=1, keepdims=True)

    @pl.when(b == 0)
    def _():
        s_o[...] = s_acc
        q_o[...] = q_acc

    @pl.when(b != 0)
    def _():
        s_o[...] += s_acc
        q_o[...] += q_acc


def _convmid(Y, s, q, g, bb, W, n):
    B, _, O, P = Y.shape
    O2 = W.shape[0]
    return pl.pallas_call(
        functools.partial(_convmid_body, P, O, O2, n),
        grid=(B,),
        in_specs=[
            pl.BlockSpec((1, _K, O, P), lambda b: (b, 0, 0, 0)),
            pl.BlockSpec((O, 1), lambda b: (0, 0)),
            pl.BlockSpec((O, 1), lambda b: (0, 0)),
            pl.BlockSpec((O, 1), lambda b: (0, 0)),
            pl.BlockSpec((O, 1), lambda b: (0, 0)),
            pl.BlockSpec((O2, O), lambda b: (0, 0)),
        ],
        out_specs=[
            pl.BlockSpec((1, _K, O2, P), lambda b: (b, 0, 0, 0)),
            pl.BlockSpec((O2, 1), lambda b: (0, 0)),
            pl.BlockSpec((O2, 1), lambda b: (0, 0)),
        ],
        out_shape=[
            jax.ShapeDtypeStruct((B, _K, O2, P), _F32),
            jax.ShapeDtypeStruct((O2, 1), _F32),
            jax.ShapeDtypeStruct((O2, 1), _F32),
        ],
    )(Y, s, q, g, bb, W)


def _ecfinal_body(P, O, n3, nsc, fuse, O1, *refs):
    if fuse:
        (y_ref, s_ref, q_ref, g_ref, b_ref, sc_ref, ss_ref, sq_ref,
         scg_ref, scb_ref, f1_ref, fw1_ref, fw2_ref,
         fts_o, G_o, fs_o, fq_o) = refs
    else:
        (y_ref, s_ref, q_ref, g_ref, b_ref, sc_ref, ss_ref, sq_ref,
         scg_ref, scb_ref, fts_o) = refs
    b = pl.program_id(0)
    m = s_ref[...] / n3
    var = q_ref[...] / n3 - m * m
    inv = 1.0 / jnp.sqrt(var + _EPS)
    facc = jnp.zeros((O, P), _F32)
    for k in range(_K):
        yk = y_ref[0, k]
        facc += jnp.maximum(g_ref[...] * (yk - m) * inv + b_ref[...], 0.0)
    f = facc / float(_K)
    msc = ss_ref[...] / nsc
    vsc = sq_ref[...] / nsc - msc * msc
    scn = (scg_ref[...] * (sc_ref[0] - msc) / jnp.sqrt(vsc + _EPS)
           + scb_ref[...])
    out = jnp.maximum(scn + f, 0.0)
    fts_o[0] = out
    if fuse:
        G = _dot(fw1_ref[...], f1_ref[0]) + _dot(fw2_ref[...], out)
        G_o[0] = G
        fsp = jnp.sum(G, axis=1, keepdims=True)
        fqp = jnp.sum(G * G, axis=1, keepdims=True)

        @pl.when(b == 0)
        def _():
            fs_o[...] = fsp
            fq_o[...] = fqp

        @pl.when(b != 0)
        def _():
            fs_o[...] += fsp
            fq_o[...] += fqp


def _ecfinal(Y3, s3, q3, g3, b3, sc, ss, sq, scg, scb, n3, nsc, fuse=None):
    B, _, O, P = Y3.shape
    ins = [Y3, s3, q3, g3, b3, sc, ss, sq, scg, scb]
    in_specs = [
        pl.BlockSpec((1, _K, O, P), lambda b: (b, 0, 0, 0)),
        pl.BlockSpec((O, 1), lambda b: (0, 0)),
        pl.BlockSpec((O, 1), lambda b: (0, 0)),
        pl.BlockSpec((O, 1), lambda b: (0, 0)),
        pl.BlockSpec((O, 1), lambda b: (0, 0)),
        pl.BlockSpec((1, O, P), lambda b: (b, 0, 0)),
        pl.BlockSpec((O, 1), lambda b: (0, 0)),
        pl.BlockSpec((O, 1), lambda b: (0, 0)),
        pl.BlockSpec((O, 1), lambda b: (0, 0)),
        pl.BlockSpec((O, 1), lambda b: (0, 0)),
    ]
    out_specs = [pl.BlockSpec((1, O, P), lambda b: (b, 0, 0))]
    out_shape = [jax.ShapeDtypeStruct((B, O, P), _F32)]
    O1 = 0
    if fuse is not None:
        f1, fw1, fw2 = fuse
        O1 = f1.shape[1]
        OF = fw1.shape[0]
        ins += [f1, fw1, fw2]
        in_specs += [
            pl.BlockSpec((1, O1, P), lambda b: (b, 0, 0)),
            pl.BlockSpec((OF, O1), lambda b: (0, 0)),
            pl.BlockSpec((OF, O), lambda b: (0, 0)),
        ]
        out_specs += [
            pl.BlockSpec((1, OF, P), lambda b: (b, 0, 0)),
            pl.BlockSpec((OF, 1), lambda b: (0, 0)),
            pl.BlockSpec((OF, 1), lambda b: (0, 0)),
        ]
        out_shape += [
            jax.ShapeDtypeStruct((B, OF, P), _F32),
            jax.ShapeDtypeStruct((OF, 1), _F32),
            jax.ShapeDtypeStruct((OF, 1), _F32),
        ]
    return pl.pallas_call(
        functools.partial(_ecfinal_body, P, O, n3, nsc,
                          fuse is not None, O1),
        grid=(B,), in_specs=in_specs, out_specs=out_specs,
        out_shape=out_shape,
    )(*ins)


def _head_body(P, OF, n, G_ref, fs_ref, fq_ref, fg_ref, fb_ref, cnt_ref,
               w1_ref, b1_ref, w2_ref, b2_ref, out_o):
    m = fs_ref[...] / n
    var = fq_ref[...] / n - m * m
    Gb = jnp.maximum(fg_ref[...] * (G_ref[0] - m) / jnp.sqrt(var + _EPS)
                     + fb_ref[...], 0.0)
    ones = jnp.ones((1, P), _F32)
    pooled = _dot_t(ones, Gb) / cnt_ref[0, 0, 0]
    h = jnp.maximum(_dot_t(pooled, w1_ref[...]) + b1_ref[...], 0.0)
    out_o[0] = _dot_t(h, w2_ref[...]) + b2_ref[...]


def _head(G, fs, fq, fg, fb, counts, w1, b1, w2, b2, n):
    B, OF, P = G.shape
    NO = w2.shape[0]
    return pl.pallas_call(
        functools.partial(_head_body, P, OF, n),
        grid=(B,),
        in_specs=[
            pl.BlockSpec((1, OF, P), lambda b: (b, 0, 0)),
            pl.BlockSpec((OF, 1), lambda b: (0, 0)),
            pl.BlockSpec((OF, 1), lambda b: (0, 0)),
            pl.BlockSpec((OF, 1), lambda b: (0, 0)),
            pl.BlockSpec((OF, 1), lambda b: (0, 0)),
            pl.BlockSpec((1, 1, 1), lambda b: (b, 0, 0)),
            pl.BlockSpec((128, 128), lambda b: (0, 0)),
            pl.BlockSpec((1, 128), lambda b: (0, 0)),
            pl.BlockSpec((NO, 128), lambda b: (0, 0)),
            pl.BlockSpec((1, NO), lambda b: (0, 0)),
        ],
        out_specs=pl.BlockSpec((1, 1, NO), lambda b: (b, 0, 0)),
        out_shape=jax.ShapeDtypeStruct((B, 1, NO), _F32),
    )(G, fs, fq, fg, fb, counts, w1, b1, w2, b2)


def _edge_conv(pts, cshift, feats, f16, p, nv, B, P, bn=None):
    Din = 4 * nv
    Dy = 16 if Din <= 16 else Din
    res = _knn(pts, cshift, feats, f16, nv, Dy, bn=bn)
    if bn is not None:
        idx, y, fts_in = res
    else:
        idx, y, fts_in = res[0], res[1], feats
    yg = _sc_gather(y.reshape(B * P, Dy), idx.reshape(B * _K * P), Dy)
    yg = yg.reshape(B, _K * P, Dy)

    W0 = p['convW'][0]
    Wx, Wz = W0[:, :Din], W0[:, Din:]
    cols = jnp.asarray([4 * v + a for a in range(4) for v in range(nv)])
    A = Wx - Wz
    Bz = Wz[:, cols]
    n_edges = float(B * P * _K)
    n_pts = float(B * P)
    Y1, scv, s1, q1, ss, sq = _conv1(fts_in, f16, yg, A, Bz, p['scW'], nv)
    g = [gg.reshape(-1, 1) for gg in p['bng']]
    bb = [bbb.reshape(-1, 1) for bbb in p['bnb']]
    Y2, s2, q2 = _convmid(Y1, s1, q1, g[0], bb[0], p['convW'][1], n_edges)
    Y3, s3, q3 = _convmid(Y2, s2, q2, g[1], bb[1], p['convW'][2], n_edges)
    return (Y3, s3, q3, g[2], bb[2], scv, ss, sq,
            p['scg'].reshape(-1, 1), p['scb'].reshape(-1, 1),
            n_edges, n_pts), fts_in


def kernel(points, features, frames, params):
    B, _, P = points.shape
    frames_r = frames.reshape(B, P, 16)
    (pts_m, feats_m, cshift, f16, fsum, fsq, counts) = _prep(
        points, features, frames_r)

    n_pts = float(B * P)
    bn0 = (fsum, fsq, params['bn_fts_g'].reshape(-1, 1),
           params['bn_fts_b'].reshape(-1, 1), n_pts)
    args1, _ = _edge_conv(pts_m, cshift, feats_m, f16, params['ec1'],
                          2, B, P, bn=bn0)
    fts1 = _ecfinal(*args1[:10], args1[10], args1[11])[0]

    args2, _ = _edge_conv(fts1, cshift, fts1, f16, params['ec2'],
                          8, B, P, bn=None)
    fuse = (fts1, params['fusionW'][:, :fts1.shape[1]],
            params['fusionW'][:, fts1.shape[1]:])
    fts2, G, fs, fq = _ecfinal(*args2[:10], args2[10], args2[11], fuse=fuse)

    out = _head(G, fs, fq, params['fusion_g'].reshape(-1, 1),
                params['fusion_b'].reshape(-1, 1), counts,
                params['fc1W'], params['fc1b'].reshape(1, -1),
                params['fc2W'], params['fc2b'].reshape(1, -1), n_pts)
    return out.reshape(B, -1)
